# baseline (speedup 1.0000x reference)
import functools

import jax
import jax.numpy as jnp
from jax.experimental import pallas as pl

_K = 10
_V = 8192
_ROWS = 32
_S = 8
_B = 128
_NEG = float("-inf")


_T = _V // 128
_F = 3


def _topk_body(x, iota):
    vals = []
    idxs = []
    for _ in range(_K):
        m = jnp.max(x, axis=1, keepdims=True)
        i = jnp.min(jnp.where(x == m, iota, _V), axis=1, keepdims=True)
        vals.append(m)
        idxs.append(i)
        x = jnp.where(iota == i, _NEG, x)
    return jnp.concatenate(vals, axis=1), jnp.concatenate(idxs, axis=1)


def _shift(x):
    m = jnp.max(x, axis=1, keepdims=True)
    shifted = x - m
    s = jnp.sum(jnp.exp(shifted), axis=1, keepdims=True)
    return shifted - jnp.log(s)


def _topk_kernel(x_ref, vals_ref, idx_ref, *, shift):
    x = x_ref[...]
    if shift:
        mx = jnp.max(x, axis=1, keepdims=True)
        s = jnp.sum(jnp.exp(x - mx), axis=1, keepdims=True)
        logs = jnp.log(s)

    lane = jax.lax.broadcasted_iota(jnp.int32, (x.shape[0], 128), 1)
    neg = jnp.full(lane.shape, _NEG, jnp.float32)
    zero = jnp.zeros(lane.shape, jnp.int32)
    v1, v2, v3 = neg, neg, neg
    i1, i2, i3 = zero, zero, zero
    for t in range(_T):
        v = x[:, t * 128:(t + 1) * 128]
        if shift:
            v = (v - mx) - logs
        gt1 = v > v1
        gt2 = v > v2
        gt3 = v > v3
        nv2 = jnp.where(gt1, v1, jnp.where(gt2, v, v2))
        ni2 = jnp.where(gt1, i1, jnp.where(gt2, t, i2))
        v3 = jnp.where(gt2, v2, jnp.where(gt3, v, v3))
        i3 = jnp.where(gt2, i2, jnp.where(gt3, t, i3))
        v1 = jnp.where(gt1, v, v1)
        i1 = jnp.where(gt1, t, i1)
        v2, i2 = nv2, ni2
    cand_v = [v1, v2, v3]
    cand_i = [i1 * 128 + lane, i2 * 128 + lane, i3 * 128 + lane]

    consumed = [jnp.zeros(lane.shape, jnp.int32) for _ in range(_F)]
    vals = []
    idxs = []
    for _ in range(_K):
        m = cand_v[0]
        for f in range(1, _F):
            m = jnp.maximum(m, cand_v[f])
        m = jnp.max(m, axis=1, keepdims=True)
        li = jnp.full(lane.shape, _V, jnp.int32)
        for f in range(_F):
            li = jnp.minimum(li, jnp.where(cand_v[f] == m, cand_i[f], _V))
        li = jnp.min(li, axis=1, keepdims=True)
        vals.append(m)
        idxs.append(li)
        for f in range(_F):
            hit = cand_i[f] == li
            cand_v[f] = jnp.where(hit, _NEG, cand_v[f])
            consumed[f] = consumed[f] + hit.astype(jnp.int32)
    vals_ref[...] = jnp.concatenate(vals, axis=1)
    idx_ref[...] = jnp.concatenate(idxs, axis=1)

    used = consumed[0]
    for f in range(1, _F):
        used = used + consumed[f]
    need_fallback = jnp.any(used >= _F)

    @pl.when(need_fallback)
    def _():
        x = x_ref[...]
        if shift:
            x = _shift(x)
        iota = jax.lax.broadcasted_iota(jnp.int32, x.shape, 1)
        fvals, fidx = _topk_body(x, iota)
        vals_ref[...] = fvals
        idx_ref[...] = fidx


def _rowwise_topk(x, shift):
    n = x.shape[0]
    return pl.pallas_call(
        functools.partial(_topk_kernel, shift=shift),
        grid=(n // _ROWS,),
        in_specs=[pl.BlockSpec((_ROWS, _V), lambda i: (i, 0))],
        out_specs=[
            pl.BlockSpec((_ROWS, _K), lambda i: (i, 0)),
            pl.BlockSpec((_ROWS, _K), lambda i: (i, 0)),
        ],
        out_shape=[
            jax.ShapeDtypeStruct((n, _K), jnp.float32),
            jax.ShapeDtypeStruct((n, _K), jnp.int32),
        ],
    )(x)


def _beam_kernel(scores_ref, idx_ref, sc_ref, struc_ref, seq_ref, res_ref):
    scores = scores_ref[...]
    idxs = idx_ref[...]
    seq_scores = scores[0]
    seq = [idxs[0]]

    iota_r = jax.lax.broadcasted_iota(jnp.int32, (_K, _K, _B), 0)
    iota_c = jax.lax.broadcasted_iota(jnp.int32, (_K, _K, _B), 1)
    lin = iota_r * _K + iota_c

    for i in range(1, _S):
        comb = seq_scores[:, None, :] + scores[i][None, :, :]
        new_scores = []
        coords = []
        for _ in range(_K):
            m = jnp.max(jnp.max(comb, axis=0, keepdims=True), axis=1,
                        keepdims=True)
            sel = jnp.where(comb == m, lin, _K * _K)
            coord = jnp.min(jnp.min(sel, axis=0, keepdims=True), axis=1,
                            keepdims=True)
            new_scores.append(m[0])
            coords.append(coord[0])
            comb = jnp.where(lin == coord, _NEG, comb)
        seq_scores = jnp.concatenate(new_scores, axis=0)
        coord = jnp.concatenate(coords, axis=0)
        r = coord // _K
        c = coord % _K
        jj = jax.lax.broadcasted_iota(jnp.int32, (_K, _K, _B), 1)
        mask_r = (r[:, None, :] == jj).astype(jnp.int32)
        mask_c = (c[:, None, :] == jj).astype(jnp.int32)
        seq = [jnp.sum(s[None, :, :] * mask_r, axis=1) for s in seq]
        seq.append(jnp.sum(idxs[i][None, :, :] * mask_c, axis=1))

    seq_arr = jnp.stack(seq, axis=0)
    seq_ref[...] = seq_arr

    struc = struc_ref[...][0]
    lr = (struc == 1)[None, :]
    sc = (struc == 0)[None, :]
    zero = jnp.zeros((_K, _B), jnp.int32)
    res = [jnp.where(lr, s, zero) for s in seq]
    res[0] = res[0] + jnp.where(sc, sc_ref[...], zero)
    res_ref[...] = jnp.stack(res, axis=0)


def _beam(scores, idxs, sc_idx, struc):
    return pl.pallas_call(
        _beam_kernel,
        out_shape=[
            jax.ShapeDtypeStruct((_S, _K, _B), jnp.int32),
            jax.ShapeDtypeStruct((_S, _K, _B), jnp.int32),
        ],
    )(scores, idxs, sc_idx, struc)


def kernel(pred_char_struc, pred_sc_logits, pred_lr_compo_logits):
    _, sc_idx = _rowwise_topk(pred_sc_logits, shift=False)
    flat = pred_lr_compo_logits.reshape(_B * _S, _V)
    cvals, cidx = _rowwise_topk(flat, shift=True)
    scores = jnp.transpose(cvals.reshape(_B, _S, _K), (1, 2, 0))
    idxs = jnp.transpose(cidx.reshape(_B, _S, _K), (1, 2, 0))
    sc_t = jnp.transpose(sc_idx, (1, 0))
    struc = jnp.broadcast_to(pred_char_struc[None, :], (_ROWS, _B))
    seq, res = _beam(scores, idxs, sc_t, struc)
    pred_lr_compo_seq = jnp.transpose(seq, (2, 1, 0))
    pred_results = jnp.transpose(res, (2, 1, 0))
    return sc_idx, pred_lr_compo_seq, pred_results


# speedup vs baseline: 1.1808x; 1.1808x over previous
import functools

import jax
import jax.numpy as jnp
from jax.experimental import pallas as pl

_K = 10
_V = 8192
_ROWS = 64
_S = 8
_B = 128
_NEG = float("-inf")


_T = _V // 128
_F = 3


def _topk_body(x, iota):
    vals = []
    idxs = []
    for _ in range(_K):
        m = jnp.max(x, axis=1, keepdims=True)
        i = jnp.min(jnp.where(x == m, iota, _V), axis=1, keepdims=True)
        vals.append(m)
        idxs.append(i)
        x = jnp.where(iota == i, _NEG, x)
    return jnp.concatenate(vals, axis=1), jnp.concatenate(idxs, axis=1)


def _shift(x):
    m = jnp.max(x, axis=1, keepdims=True)
    shifted = x - m
    s = jnp.sum(jnp.exp(shifted), axis=1, keepdims=True)
    return shifted - jnp.log(s)


def _topk_kernel(x_ref, vals_ref, idx_ref, *, shift):
    x = x_ref[...]
    if shift:
        mx = jnp.max(x, axis=1, keepdims=True)
        s = jnp.sum(jnp.exp(x - mx), axis=1, keepdims=True)
        logs = jnp.log(s)

    lane = jax.lax.broadcasted_iota(jnp.int32, (x.shape[0], 128), 1)
    neg = jnp.full(lane.shape, _NEG, jnp.float32)
    zero = jnp.zeros(lane.shape, jnp.int32)
    v1, v2, v3 = neg, neg, neg
    i1, i2, i3 = zero, zero, zero
    for t in range(_T):
        v = x[:, t * 128:(t + 1) * 128]
        if shift:
            v = (v - mx) - logs
        gt1 = v > v1
        gt2 = v > v2
        gt3 = v > v3
        nv2 = jnp.where(gt1, v1, jnp.where(gt2, v, v2))
        ni2 = jnp.where(gt1, i1, jnp.where(gt2, t, i2))
        v3 = jnp.where(gt2, v2, jnp.where(gt3, v, v3))
        i3 = jnp.where(gt2, i2, jnp.where(gt3, t, i3))
        v1 = jnp.where(gt1, v, v1)
        i1 = jnp.where(gt1, t, i1)
        v2, i2 = nv2, ni2
    cand_v = [v1, v2, v3]
    cand_i = [i1 * 128 + lane, i2 * 128 + lane, i3 * 128 + lane]

    consumed = [jnp.zeros(lane.shape, jnp.int32) for _ in range(_F)]
    vals = []
    idxs = []
    for _ in range(_K):
        m = cand_v[0]
        for f in range(1, _F):
            m = jnp.maximum(m, cand_v[f])
        m = jnp.max(m, axis=1, keepdims=True)
        li = jnp.full(lane.shape, _V, jnp.int32)
        for f in range(_F):
            li = jnp.minimum(li, jnp.where(cand_v[f] == m, cand_i[f], _V))
        li = jnp.min(li, axis=1, keepdims=True)
        vals.append(m)
        idxs.append(li)
        for f in range(_F):
            hit = cand_i[f] == li
            cand_v[f] = jnp.where(hit, _NEG, cand_v[f])
            consumed[f] = consumed[f] + hit.astype(jnp.int32)
    vals_ref[...] = jnp.concatenate(vals, axis=1)
    idx_ref[...] = jnp.concatenate(idxs, axis=1)

    used = consumed[0]
    for f in range(1, _F):
        used = used + consumed[f]
    need_fallback = jnp.any(used >= _F)

    @pl.when(need_fallback)
    def _():
        x = x_ref[...]
        if shift:
            x = _shift(x)
        iota = jax.lax.broadcasted_iota(jnp.int32, x.shape, 1)
        fvals, fidx = _topk_body(x, iota)
        vals_ref[...] = fvals
        idx_ref[...] = fidx


def _rowwise_topk(x, shift):
    n = x.shape[0]
    return pl.pallas_call(
        functools.partial(_topk_kernel, shift=shift),
        grid=(n // _ROWS,),
        in_specs=[pl.BlockSpec((_ROWS, _V), lambda i: (i, 0))],
        out_specs=[
            pl.BlockSpec((_ROWS, _K), lambda i: (i, 0)),
            pl.BlockSpec((_ROWS, _K), lambda i: (i, 0)),
        ],
        out_shape=[
            jax.ShapeDtypeStruct((n, _K), jnp.float32),
            jax.ShapeDtypeStruct((n, _K), jnp.int32),
        ],
    )(x)


def _beam_kernel(scores_ref, idx_ref, sc_ref, struc_ref, seq_ref, res_ref):
    scores = scores_ref[...]
    idxs = idx_ref[...]
    seq_scores = scores[0]
    seq = [idxs[0]]

    iota_r = jax.lax.broadcasted_iota(jnp.int32, (_K, _K, _B), 0)
    iota_c = jax.lax.broadcasted_iota(jnp.int32, (_K, _K, _B), 1)
    lin = iota_r * _K + iota_c

    for i in range(1, _S):
        comb = seq_scores[:, None, :] + scores[i][None, :, :]
        new_scores = []
        coords = []
        for _ in range(_K):
            m = jnp.max(jnp.max(comb, axis=0, keepdims=True), axis=1,
                        keepdims=True)
            sel = jnp.where(comb == m, lin, _K * _K)
            coord = jnp.min(jnp.min(sel, axis=0, keepdims=True), axis=1,
                            keepdims=True)
            new_scores.append(m[0])
            coords.append(coord[0])
            comb = jnp.where(lin == coord, _NEG, comb)
        seq_scores = jnp.concatenate(new_scores, axis=0)
        coord = jnp.concatenate(coords, axis=0)
        r = coord // _K
        c = coord % _K
        jj = jax.lax.broadcasted_iota(jnp.int32, (_K, _K, _B), 1)
        mask_r = (r[:, None, :] == jj).astype(jnp.int32)
        mask_c = (c[:, None, :] == jj).astype(jnp.int32)
        seq = [jnp.sum(s[None, :, :] * mask_r, axis=1) for s in seq]
        seq.append(jnp.sum(idxs[i][None, :, :] * mask_c, axis=1))

    seq_arr = jnp.stack(seq, axis=0)
    seq_ref[...] = seq_arr

    struc = struc_ref[...][0]
    lr = (struc == 1)[None, :]
    sc = (struc == 0)[None, :]
    zero = jnp.zeros((_K, _B), jnp.int32)
    res = [jnp.where(lr, s, zero) for s in seq]
    res[0] = res[0] + jnp.where(sc, sc_ref[...], zero)
    res_ref[...] = jnp.stack(res, axis=0)


def _beam(scores, idxs, sc_idx, struc):
    return pl.pallas_call(
        _beam_kernel,
        out_shape=[
            jax.ShapeDtypeStruct((_S, _K, _B), jnp.int32),
            jax.ShapeDtypeStruct((_S, _K, _B), jnp.int32),
        ],
    )(scores, idxs, sc_idx, struc)


def kernel(pred_char_struc, pred_sc_logits, pred_lr_compo_logits):
    _, sc_idx = _rowwise_topk(pred_sc_logits, shift=False)
    flat = pred_lr_compo_logits.reshape(_B * _S, _V)
    cvals, cidx = _rowwise_topk(flat, shift=True)
    scores = jnp.transpose(cvals.reshape(_B, _S, _K), (1, 2, 0))
    idxs = jnp.transpose(cidx.reshape(_B, _S, _K), (1, 2, 0))
    sc_t = jnp.transpose(sc_idx, (1, 0))
    struc = jnp.broadcast_to(pred_char_struc[None, :], (_ROWS, _B))
    seq, res = _beam(scores, idxs, sc_t, struc)
    pred_lr_compo_seq = jnp.transpose(seq, (2, 1, 0))
    pred_results = jnp.transpose(res, (2, 1, 0))
    return sc_idx, pred_lr_compo_seq, pred_results
